# baseline (speedup 1.0000x reference)
import functools

import jax
import jax.numpy as jnp
from jax import lax
from jax.experimental import pallas as pl
from jax.experimental.pallas import tpu as pltpu
from jax.experimental.pallas import tpu_sc as plsc

K_STATIC = 64
NC, NS, L = 2, 16, 16
NW = NC * NS
FETCH = 4


def _encode_kernel(x_ref, w_ref, b_ref, z_ref, cm_ref):
    i = pl.program_id(0)
    z = lax.dot_general(
        x_ref[...], w_ref[...], (((1,), (1,)), ((), ())),
        preferred_element_type=jnp.float32)
    z = jnp.maximum(z + b_ref[...], 0.0)
    z_ref[...] = z
    nsl = z.shape[1] // 128
    cm = z[:, 0:128]
    for a in range(1, nsl):
        cm = jnp.maximum(cm, z[:, a * 128:(a + 1) * 128])

    @pl.when(i == 0)
    def _():
        cm_ref[...] = cm

    @pl.when(i != 0)
    def _():
        cm_ref[...] = jnp.maximum(cm_ref[...], cm)


def _bounds_kernel(k_ref, cm_ref, t0_ref, hi_ref, lo_s, hi_s):
    k = k_ref[0]
    bits = lax.bitcast_convert_type(cm_ref[...], jnp.int32)
    rowmax = jnp.max(bits, axis=1, keepdims=True)
    lo_s[...] = jnp.zeros_like(rowmax)
    hi_s[...] = rowmax + 1

    def body(_, carry):
        lo, hi = lo_s[...], hi_s[...]
        mid = lax.shift_right_logical(lo + hi, 1)
        c = jnp.sum((bits >= mid).astype(jnp.int32), axis=1, keepdims=True)
        ge = c >= k
        lo_s[...] = jnp.where(ge, mid, lo)
        hi_s[...] = jnp.where(ge, hi, mid)
        return carry

    lax.fori_loop(0, 18, body, 0)
    t0_ref[...] = jnp.broadcast_to(jnp.minimum(lo_s[...], rowmax),
                                   t0_ref.shape)
    hi_ref[...] = jnp.broadcast_to(rowmax + 1, hi_ref.shape)


def _sc_select(z_ref, t0_ref, hi0_ref, k_ref, out_ref,
               buf, cand, t0v, hiv, kv, outv, sem, *, width, rpw):
    wid = lax.axis_index("s") * NC + lax.axis_index("c")
    base = wid * rpw

    pltpu.async_copy(
        t0_ref.at[pl.ds(pl.multiple_of(base * L, 8), rpw * L)],
        t0v, sem).wait()
    pltpu.async_copy(
        hi0_ref.at[pl.ds(pl.multiple_of(base * L, 8), rpw * L)],
        hiv, sem).wait()
    pltpu.async_copy(k_ref.at[pl.ds(0, L)], kv, sem).wait()
    k_v = kv[...]
    lane = lax.iota(jnp.int32, L)
    ones_v = jnp.full((L,), 1, jnp.int32)
    zero_v = jnp.full((L,), 0, jnp.int32)

    def row_body(r, carry):
        @pl.when((r & (FETCH - 1)) == 0)
        def _():
            pltpu.async_copy(
                z_ref.at[pl.ds(pl.multiple_of((base + r) * width, 8),
                               FETCH * width)],
                buf, sem).wait()

        off = (r & (FETCH - 1)) * width
        t0spl = t0v[pl.ds(r * L, L)]
        hispl = hiv[pl.ds(r * L, L)]

        seg = width // 4
        rs = seg + L

        def fbody(i, cs):
            c0, c1, c2, c3 = cs
            v0 = buf[pl.ds(off + i * L, L)]
            v1 = buf[pl.ds(off + seg + i * L, L)]
            v2 = buf[pl.ds(off + 2 * seg + i * L, L)]
            v3 = buf[pl.ds(off + 3 * seg + i * L, L)]
            b0 = lax.bitcast_convert_type(v0, jnp.int32)
            b1 = lax.bitcast_convert_type(v1, jnp.int32)
            b2 = lax.bitcast_convert_type(v2, jnp.int32)
            b3 = lax.bitcast_convert_type(v3, jnp.int32)
            m0 = b0 >= t0spl
            m1 = b1 >= t0spl
            m2 = b2 >= t0spl
            m3 = b3 >= t0spl
            plsc.store_compressed(cand.at[pl.ds(c0, L)], b0, mask=m0)
            plsc.store_compressed(cand.at[pl.ds(rs + c1, L)], b1, mask=m1)
            plsc.store_compressed(cand.at[pl.ds(2 * rs + c2, L)], b2,
                                  mask=m2)
            plsc.store_compressed(cand.at[pl.ds(3 * rs + c3, L)], b3,
                                  mask=m3)
            return (c0 + plsc.all_reduce_population_count(m0)[0],
                    c1 + plsc.all_reduce_population_count(m1)[0],
                    c2 + plsc.all_reduce_population_count(m2)[0],
                    c3 + plsc.all_reduce_population_count(m3)[0])

        c0, c1, c2, c3 = plsc.parallel_loop(
            0, seg // L, unroll=4,
            carry=(jnp.int32(0), jnp.int32(0), jnp.int32(0),
                   jnp.int32(0)))(fbody)

        neg = jnp.full((L,), -1, jnp.int32)
        cand[pl.ds(c0, L)] = neg
        cand[pl.ds(rs + c1, L)] = neg
        cand[pl.ds(2 * rs + c2, L)] = neg
        cand[pl.ds(3 * rs + c3, L)] = neg
        end = lax.shift_right_logical(c0 + L - 1, 4) * L
        for s, cs_ in ((1, c1), (2, c2), (3, c3)):
            ks = lax.shift_right_logical(cs_ + L - 1, 4)
            base_s = s * rs

            def copyb(i, e):
                cand[pl.ds(e + i * L, L)] = cand[pl.ds(base_s + i * L, L)]
                return e

            lax.fori_loop(0, ks, copyb, end)
            end = end + ks * L
        nv = lax.shift_right_logical(end, 4)

        def bis_body(_, carry):
            lo, hi, t, found = carry
            mid = lax.shift_right_logical(lo + hi, 1)

            def cb(i, acc):
                cv = cand[pl.ds(i * L, L)]
                return acc + plsc.all_reduce_population_count(cv >= mid)

            c = plsc.parallel_loop(0, nv, unroll=4, carry=zero_v)(cb)
            upd = found == 0
            ge = c >= k_v
            lo = jnp.where(jnp.logical_and(upd, ge), mid, lo)
            hi = jnp.where(
                jnp.logical_and(upd, jnp.logical_not(ge)), mid, hi)
            hit = jnp.logical_and(upd, c == k_v)
            closed = jnp.logical_and(upd, (hi - lo) <= 1)
            t = jnp.where(hit, mid, jnp.where(closed, lo, t))
            found = jnp.where(jnp.logical_or(hit, closed), ones_v, found)
            return lo, hi, t, found

        _, _, t_v, _ = lax.fori_loop(
            0, 32, bis_body, (t0spl, hispl, zero_v, zero_v))

        j = r & (L - 1)
        outv[...] = jnp.where(lane == j, t_v, outv[...])

        @pl.when(j == L - 1)
        def _():
            pltpu.async_copy(
                outv,
                out_ref.at[pl.ds(
                    pl.multiple_of(base + ((r >> 4) << 4), 8), L)],
                sem).wait()

        return carry

    lax.fori_loop(0, rpw, row_body, 0)


def _mask_decode_kernel(t_ref, z_ref, w_ref, db_ref, zm_ref, xhat_ref):
    i = pl.program_id(0)
    z = z_ref[...]
    bits = lax.bitcast_convert_type(z, jnp.int32)
    zm = jnp.where(bits >= t_ref[...], z, 0.0)
    zm_ref[...] = zm
    part = lax.dot_general(
        zm, w_ref[...], (((1,), (0,)), ((), ())),
        preferred_element_type=jnp.float32)

    @pl.when(i == 0)
    def _():
        xhat_ref[...] = part + db_ref[...]

    @pl.when(i != 0)
    def _():
        xhat_ref[...] = xhat_ref[...] + part


def kernel(x, W_enc, b_enc, dec_bias, topk):
    B, d_in = x.shape
    width = W_enc.shape[0]
    wb1 = min(512, width)
    wb2 = min(256, width)
    rb = min(512, B)
    rpw = B // NW

    b2 = b_enc.reshape(1, width)
    db2 = dec_bias.reshape(1, d_in)
    k_eff = jnp.clip(jnp.asarray(topk, jnp.int32), 0, K_STATIC)
    k1 = k_eff.reshape(1)

    z, cm = pl.pallas_call(
        _encode_kernel,
        grid=(width // wb1,),
        in_specs=[
            pl.BlockSpec((B, d_in), lambda i: (0, 0)),
            pl.BlockSpec((wb1, d_in), lambda i: (i, 0)),
            pl.BlockSpec((1, wb1), lambda i: (0, i)),
        ],
        out_specs=[
            pl.BlockSpec((B, wb1), lambda i: (0, i)),
            pl.BlockSpec((B, 128), lambda i: (0, 0)),
        ],
        out_shape=[
            jax.ShapeDtypeStruct((B, width), jnp.float32),
            jax.ShapeDtypeStruct((B, 128), jnp.float32),
        ],
    )(x, W_enc, b2)

    t0, hi0 = pl.pallas_call(
        _bounds_kernel,
        grid_spec=pltpu.PrefetchScalarGridSpec(
            num_scalar_prefetch=1,
            grid=(B // rb,),
            in_specs=[pl.BlockSpec((rb, 128), lambda i, k: (i, 0))],
            out_specs=[
                pl.BlockSpec((rb, L), lambda i, k: (i, 0)),
                pl.BlockSpec((rb, L), lambda i, k: (i, 0)),
            ],
            scratch_shapes=[
                pltpu.VMEM((rb, 1), jnp.int32),
                pltpu.VMEM((rb, 1), jnp.int32),
            ],
        ),
        out_shape=[
            jax.ShapeDtypeStruct((B, L), jnp.int32),
            jax.ShapeDtypeStruct((B, L), jnp.int32),
        ],
    )(k1, cm)

    karr = jnp.broadcast_to(k_eff, (L,)).astype(jnp.int32)
    mesh = plsc.VectorSubcoreMesh(
        core_axis_name="c", subcore_axis_name="s",
        num_cores=NC, num_subcores=NS)
    t_sc = pl.kernel(
        functools.partial(_sc_select, width=width, rpw=rpw),
        out_type=jax.ShapeDtypeStruct((B,), jnp.int32),
        mesh=mesh,
        scratch_types=[
            pltpu.VMEM((FETCH * width,), jnp.float32),
            pltpu.VMEM((width + 4 * L,), jnp.int32),
            pltpu.VMEM((rpw * L,), jnp.int32),
            pltpu.VMEM((rpw * L,), jnp.int32),
            pltpu.VMEM((L,), jnp.int32),
            pltpu.VMEM((L,), jnp.int32),
            pltpu.SemaphoreType.DMA,
        ],
        compiler_params=pltpu.CompilerParams(needs_layout_passes=False),
    )(z.reshape(B * width), t0.reshape(B * L), hi0.reshape(B * L), karr)

    t = jnp.where(k_eff <= 0, jnp.int32(0x7F800000), t_sc).reshape(B, 1)

    nsteps = width // wb2
    zm, xhat = pl.pallas_call(
        _mask_decode_kernel,
        grid=(nsteps,),
        in_specs=[
            pl.BlockSpec((B, 1), lambda i: (0, 0)),
            pl.BlockSpec((B, wb2), lambda i: (0, i)),
            pl.BlockSpec((wb2, d_in), lambda i: (i, 0)),
            pl.BlockSpec((1, d_in), lambda i: (0, 0)),
        ],
        out_specs=[
            pl.BlockSpec((B, wb2), lambda i: (0, i)),
            pl.BlockSpec((B, d_in), lambda i: (0, 0)),
        ],
        out_shape=[
            jax.ShapeDtypeStruct((B, width), jnp.float32),
            jax.ShapeDtypeStruct((B, d_in), jnp.float32),
        ],
    )(t, z, W_enc, db2)

    return (zm, xhat)


# speedup vs baseline: 1.3607x; 1.3607x over previous
import functools

import jax
import jax.numpy as jnp
from jax import lax
from jax.experimental import pallas as pl
from jax.experimental.pallas import tpu as pltpu
from jax.experimental.pallas import tpu_sc as plsc

K_STATIC = 64
NC, NS, L = 2, 16, 16
NW = NC * NS
FETCH = 4


def _encode_kernel(x_ref, w_ref, b_ref, z_ref, cm_ref):
    i = pl.program_id(0)
    z = lax.dot_general(
        x_ref[...], w_ref[...], (((1,), (1,)), ((), ())),
        preferred_element_type=jnp.float32)
    z = jnp.maximum(z + b_ref[...], 0.0)
    z_ref[...] = z
    nsl = z.shape[1] // 128
    cm = z[:, 0:128]
    for a in range(1, nsl):
        cm = jnp.maximum(cm, z[:, a * 128:(a + 1) * 128])

    @pl.when(i == 0)
    def _():
        cm_ref[...] = cm

    @pl.when(i != 0)
    def _():
        cm_ref[...] = jnp.maximum(cm_ref[...], cm)


def _bounds_kernel(k_ref, cm_ref, t0_ref, hi_ref, lo_s, hi_s):
    k = k_ref[0]
    bits = lax.bitcast_convert_type(cm_ref[...], jnp.int32)
    rowmax = jnp.max(bits, axis=1, keepdims=True)
    lo_s[...] = jnp.zeros_like(rowmax)
    hi_s[...] = rowmax + 1

    def body(_, carry):
        lo, hi = lo_s[...], hi_s[...]
        mid = lax.shift_right_logical(lo + hi, 1)
        c = jnp.sum((bits >= mid).astype(jnp.int32), axis=1, keepdims=True)
        ge = c >= k
        lo_s[...] = jnp.where(ge, mid, lo)
        hi_s[...] = jnp.where(ge, hi, mid)
        return carry

    lax.fori_loop(0, 18, body, 0)
    t0_ref[...] = jnp.broadcast_to(jnp.minimum(lo_s[...], rowmax),
                                   t0_ref.shape)
    hi_ref[...] = jnp.broadcast_to(rowmax + 1, hi_ref.shape)


def _sc_select(z_ref, t0_ref, hi0_ref, k_ref, out_ref,
               buf, cand, t0v, hiv, kv, outv, sem, *, width, rpw):
    wid = lax.axis_index("s") * NC + lax.axis_index("c")
    base = wid * rpw

    pltpu.async_copy(
        t0_ref.at[pl.ds(pl.multiple_of(base * L, 8), rpw * L)],
        t0v, sem).wait()
    pltpu.async_copy(
        hi0_ref.at[pl.ds(pl.multiple_of(base * L, 8), rpw * L)],
        hiv, sem).wait()
    pltpu.async_copy(k_ref.at[pl.ds(0, L)], kv, sem).wait()
    k_v = kv[...]
    lane = lax.iota(jnp.int32, L)
    ones_v = jnp.full((L,), 1, jnp.int32)
    zero_v = jnp.full((L,), 0, jnp.int32)

    def row_body(r, carry):
        @pl.when((r & (FETCH - 1)) == 0)
        def _():
            pltpu.async_copy(
                z_ref.at[pl.ds(pl.multiple_of((base + r) * width, 8),
                               FETCH * width)],
                buf, sem).wait()

        off = (r & (FETCH - 1)) * width
        t0spl = t0v[pl.ds(r * L, L)]
        hispl = hiv[pl.ds(r * L, L)]

        def fbody(i, cnt):
            v = buf[pl.ds(off + i * L, L)]
            b = lax.bitcast_convert_type(v, jnp.int32)
            m = b >= t0spl
            plsc.store_compressed(cand.at[pl.ds(cnt, L)], b, mask=m)
            return cnt + plsc.all_reduce_population_count(m)[0]

        cnt = plsc.parallel_loop(
            0, width // L, unroll=16, carry=jnp.int32(0))(fbody)
        cand[pl.ds(cnt, L)] = jnp.full((L,), -1, jnp.int32)
        nv = lax.shift_right_logical(cnt + L - 1, 4)

        def bis_body(_, carry):
            lo, hi, t, found = carry
            mid = lax.shift_right_logical(lo + hi, 1)

            def cb(i, acc):
                cv = cand[pl.ds(i * L, L)]
                return acc + plsc.all_reduce_population_count(cv >= mid)

            c = plsc.parallel_loop(0, nv, unroll=4, carry=zero_v)(cb)
            upd = found == 0
            ge = c >= k_v
            lo = jnp.where(jnp.logical_and(upd, ge), mid, lo)
            hi = jnp.where(
                jnp.logical_and(upd, jnp.logical_not(ge)), mid, hi)
            hit = jnp.logical_and(upd, c == k_v)
            closed = jnp.logical_and(upd, (hi - lo) <= 1)
            t = jnp.where(hit, mid, jnp.where(closed, lo, t))
            found = jnp.where(jnp.logical_or(hit, closed), ones_v, found)
            return lo, hi, t, found

        _, _, t_v, _ = lax.fori_loop(
            0, 32, bis_body, (t0spl, hispl, zero_v, zero_v))

        j = r & (L - 1)
        outv[...] = jnp.where(lane == j, t_v, outv[...])

        @pl.when(j == L - 1)
        def _():
            pltpu.async_copy(
                outv,
                out_ref.at[pl.ds(
                    pl.multiple_of(base + ((r >> 4) << 4), 8), L)],
                sem).wait()

        return carry

    lax.fori_loop(0, rpw, row_body, 0)


def _mask_decode_kernel(t_ref, z_ref, w_ref, db_ref, zm_ref, xhat_ref):
    i = pl.program_id(0)
    z = z_ref[...]
    bits = lax.bitcast_convert_type(z, jnp.int32)
    zm = jnp.where(bits >= t_ref[...], z, 0.0)
    zm_ref[...] = zm
    part = lax.dot_general(
        zm, w_ref[...], (((1,), (0,)), ((), ())),
        preferred_element_type=jnp.float32)

    @pl.when(i == 0)
    def _():
        xhat_ref[...] = part + db_ref[...]

    @pl.when(i != 0)
    def _():
        xhat_ref[...] = xhat_ref[...] + part


def kernel(x, W_enc, b_enc, dec_bias, topk):
    B, d_in = x.shape
    width = W_enc.shape[0]
    wb1 = min(512, width)
    wb2 = min(256, width)
    rb = min(512, B)
    rpw = B // NW

    b2 = b_enc.reshape(1, width)
    db2 = dec_bias.reshape(1, d_in)
    k_eff = jnp.clip(jnp.asarray(topk, jnp.int32), 0, K_STATIC)
    k1 = k_eff.reshape(1)

    z, cm = pl.pallas_call(
        _encode_kernel,
        grid=(width // wb1,),
        in_specs=[
            pl.BlockSpec((B, d_in), lambda i: (0, 0)),
            pl.BlockSpec((wb1, d_in), lambda i: (i, 0)),
            pl.BlockSpec((1, wb1), lambda i: (0, i)),
        ],
        out_specs=[
            pl.BlockSpec((B, wb1), lambda i: (0, i)),
            pl.BlockSpec((B, 128), lambda i: (0, 0)),
        ],
        out_shape=[
            jax.ShapeDtypeStruct((B, width), jnp.float32),
            jax.ShapeDtypeStruct((B, 128), jnp.float32),
        ],
    )(x, W_enc, b2)

    t0, hi0 = pl.pallas_call(
        _bounds_kernel,
        grid_spec=pltpu.PrefetchScalarGridSpec(
            num_scalar_prefetch=1,
            grid=(B // rb,),
            in_specs=[pl.BlockSpec((rb, 128), lambda i, k: (i, 0))],
            out_specs=[
                pl.BlockSpec((rb, L), lambda i, k: (i, 0)),
                pl.BlockSpec((rb, L), lambda i, k: (i, 0)),
            ],
            scratch_shapes=[
                pltpu.VMEM((rb, 1), jnp.int32),
                pltpu.VMEM((rb, 1), jnp.int32),
            ],
        ),
        out_shape=[
            jax.ShapeDtypeStruct((B, L), jnp.int32),
            jax.ShapeDtypeStruct((B, L), jnp.int32),
        ],
    )(k1, cm)

    karr = jnp.broadcast_to(k_eff, (L,)).astype(jnp.int32)
    mesh = plsc.VectorSubcoreMesh(
        core_axis_name="c", subcore_axis_name="s",
        num_cores=NC, num_subcores=NS)
    t_sc = pl.kernel(
        functools.partial(_sc_select, width=width, rpw=rpw),
        out_type=jax.ShapeDtypeStruct((B,), jnp.int32),
        mesh=mesh,
        scratch_types=[
            pltpu.VMEM((FETCH * width,), jnp.float32),
            pltpu.VMEM((width + 4 * L,), jnp.int32),
            pltpu.VMEM((rpw * L,), jnp.int32),
            pltpu.VMEM((rpw * L,), jnp.int32),
            pltpu.VMEM((L,), jnp.int32),
            pltpu.VMEM((L,), jnp.int32),
            pltpu.SemaphoreType.DMA,
        ],
        compiler_params=pltpu.CompilerParams(needs_layout_passes=False),
    )(z.reshape(B * width), t0.reshape(B * L), hi0.reshape(B * L), karr)

    t = jnp.where(k_eff <= 0, jnp.int32(0x7F800000), t_sc).reshape(B, 1)

    nsteps = width // wb2
    zm, xhat = pl.pallas_call(
        _mask_decode_kernel,
        grid=(nsteps,),
        in_specs=[
            pl.BlockSpec((B, 1), lambda i: (0, 0)),
            pl.BlockSpec((B, wb2), lambda i: (0, i)),
            pl.BlockSpec((wb2, d_in), lambda i: (i, 0)),
            pl.BlockSpec((1, d_in), lambda i: (0, 0)),
        ],
        out_specs=[
            pl.BlockSpec((B, wb2), lambda i: (0, i)),
            pl.BlockSpec((B, d_in), lambda i: (0, 0)),
        ],
        out_shape=[
            jax.ShapeDtypeStruct((B, width), jnp.float32),
            jax.ShapeDtypeStruct((B, d_in), jnp.float32),
        ],
    )(t, z, W_enc, db2)

    return (zm, xhat)


# speedup vs baseline: 1.5249x; 1.1207x over previous
import functools

import jax
import jax.numpy as jnp
from jax import lax
from jax.experimental import pallas as pl
from jax.experimental.pallas import tpu as pltpu
from jax.experimental.pallas import tpu_sc as plsc

K_STATIC = 64
NC, NS, L = 2, 16, 16
NW = NC * NS
FETCH = 4


def _encode_kernel(x_ref, w_ref, b_ref, z_ref, cm_ref):
    i = pl.program_id(0)
    z = lax.dot_general(
        x_ref[...], w_ref[...], (((1,), (1,)), ((), ())),
        preferred_element_type=jnp.float32)
    z = jnp.maximum(z + b_ref[...], 0.0)
    z_ref[...] = z
    nsl = z.shape[1] // 128
    cm = z[:, 0:128]
    for a in range(1, nsl):
        cm = jnp.maximum(cm, z[:, a * 128:(a + 1) * 128])

    @pl.when(i == 0)
    def _():
        cm_ref[...] = cm

    @pl.when(i != 0)
    def _():
        cm_ref[...] = jnp.maximum(cm_ref[...], cm)


def _bounds_kernel(k_ref, cm_ref, t0_ref, hi_ref, lo_s, hi_s):
    k = k_ref[0]
    bits = lax.bitcast_convert_type(cm_ref[...], jnp.int32)
    rowmax = jnp.max(bits, axis=1, keepdims=True)
    lo_s[...] = jnp.zeros_like(rowmax)
    hi_s[...] = rowmax + 1

    def body(_, carry):
        lo, hi = lo_s[...], hi_s[...]
        mid = lax.shift_right_logical(lo + hi, 1)
        c = jnp.sum((bits >= mid).astype(jnp.int32), axis=1, keepdims=True)
        ge = c >= k
        lo_s[...] = jnp.where(ge, mid, lo)
        hi_s[...] = jnp.where(ge, hi, mid)
        return carry

    lax.fori_loop(0, 18, body, 0)
    t0_ref[...] = jnp.broadcast_to(jnp.minimum(lo_s[...], rowmax),
                                   t0_ref.shape)
    hi_ref[...] = jnp.broadcast_to(rowmax + 1, hi_ref.shape)


def _sc_select(z_ref, t0_ref, hi0_ref, k_ref, out_ref,
               buf, cand, t0v, hiv, kv, outv, sem, *, width, rpw):
    wid = lax.axis_index("s") * NC + lax.axis_index("c")
    base = wid * rpw

    pltpu.async_copy(
        t0_ref.at[pl.ds(pl.multiple_of(base * L, 8), rpw * L)],
        t0v, sem).wait()
    pltpu.async_copy(
        hi0_ref.at[pl.ds(pl.multiple_of(base * L, 8), rpw * L)],
        hiv, sem).wait()
    pltpu.async_copy(k_ref.at[pl.ds(0, L)], kv, sem).wait()
    k_v = kv[...]
    lane = lax.iota(jnp.int32, L)
    ones_v = jnp.full((L,), 1, jnp.int32)
    zero_v = jnp.full((L,), 0, jnp.int32)

    def row_body(r, carry):
        @pl.when((r & (FETCH - 1)) == 0)
        def _():
            pltpu.async_copy(
                z_ref.at[pl.ds(pl.multiple_of((base + r) * width, 8),
                               FETCH * width)],
                buf, sem).wait()

        off = (r & (FETCH - 1)) * width
        t0spl = t0v[pl.ds(r * L, L)]
        hispl = hiv[pl.ds(r * L, L)]

        def fbody(i, cnt):
            v = buf[pl.ds(off + i * L, L)]
            b = lax.bitcast_convert_type(v, jnp.int32)
            m = b >= t0spl
            plsc.store_compressed(cand.at[pl.ds(cnt, L)], b, mask=m)
            return cnt + plsc.all_reduce_population_count(m)[0]

        cnt = plsc.parallel_loop(
            0, width // L, unroll=16, carry=jnp.int32(0))(fbody)
        cand[pl.ds(cnt, L)] = jnp.full((L,), -1, jnp.int32)
        nv = lax.shift_right_logical(cnt + L - 1, 4)

        def bis_cond(carry):
            _, _, _, found = carry
            return found == 0

        def bis_body(carry):
            lo, hi, t, found = carry
            mid = lax.shift_right_logical(lo + hi, 1)

            def cb(i, acc):
                cv = cand[pl.ds(i * L, L)]
                return acc + plsc.all_reduce_population_count(cv >= mid)

            c = plsc.parallel_loop(0, nv, unroll=4, carry=zero_v)(cb)
            ge = c >= k_v
            lo = jnp.where(ge, mid, lo)
            hi = jnp.where(jnp.logical_not(ge), mid, hi)
            hit = c == k_v
            closed = (hi - lo) <= 1
            t = jnp.where(hit, mid, jnp.where(closed, lo, t))
            found = jnp.where(
                jnp.logical_or(hit, closed), jnp.int32(1), found)[0]
            return lo, hi, t, found

        _, _, t_v, _ = lax.while_loop(
            bis_cond, bis_body, (t0spl, hispl, zero_v, jnp.int32(0)))

        j = r & (L - 1)
        outv[...] = jnp.where(lane == j, t_v, outv[...])

        @pl.when(j == L - 1)
        def _():
            pltpu.async_copy(
                outv,
                out_ref.at[pl.ds(
                    pl.multiple_of(base + ((r >> 4) << 4), 8), L)],
                sem).wait()

        return carry

    lax.fori_loop(0, rpw, row_body, 0)


def _mask_decode_kernel(t_ref, z_ref, w_ref, db_ref, zm_ref, xhat_ref):
    i = pl.program_id(0)
    z = z_ref[...]
    bits = lax.bitcast_convert_type(z, jnp.int32)
    zm = jnp.where(bits >= t_ref[...], z, 0.0)
    zm_ref[...] = zm
    part = lax.dot_general(
        zm, w_ref[...], (((1,), (0,)), ((), ())),
        preferred_element_type=jnp.float32)

    @pl.when(i == 0)
    def _():
        xhat_ref[...] = part + db_ref[...]

    @pl.when(i != 0)
    def _():
        xhat_ref[...] = xhat_ref[...] + part


def kernel(x, W_enc, b_enc, dec_bias, topk):
    B, d_in = x.shape
    width = W_enc.shape[0]
    wb1 = min(512, width)
    wb2 = min(256, width)
    rb = min(512, B)
    rpw = B // NW

    b2 = b_enc.reshape(1, width)
    db2 = dec_bias.reshape(1, d_in)
    k_eff = jnp.clip(jnp.asarray(topk, jnp.int32), 0, K_STATIC)
    k1 = k_eff.reshape(1)

    z, cm = pl.pallas_call(
        _encode_kernel,
        grid=(width // wb1,),
        in_specs=[
            pl.BlockSpec((B, d_in), lambda i: (0, 0)),
            pl.BlockSpec((wb1, d_in), lambda i: (i, 0)),
            pl.BlockSpec((1, wb1), lambda i: (0, i)),
        ],
        out_specs=[
            pl.BlockSpec((B, wb1), lambda i: (0, i)),
            pl.BlockSpec((B, 128), lambda i: (0, 0)),
        ],
        out_shape=[
            jax.ShapeDtypeStruct((B, width), jnp.float32),
            jax.ShapeDtypeStruct((B, 128), jnp.float32),
        ],
    )(x, W_enc, b2)

    t0, hi0 = pl.pallas_call(
        _bounds_kernel,
        grid_spec=pltpu.PrefetchScalarGridSpec(
            num_scalar_prefetch=1,
            grid=(B // rb,),
            in_specs=[pl.BlockSpec((rb, 128), lambda i, k: (i, 0))],
            out_specs=[
                pl.BlockSpec((rb, L), lambda i, k: (i, 0)),
                pl.BlockSpec((rb, L), lambda i, k: (i, 0)),
            ],
            scratch_shapes=[
                pltpu.VMEM((rb, 1), jnp.int32),
                pltpu.VMEM((rb, 1), jnp.int32),
            ],
        ),
        out_shape=[
            jax.ShapeDtypeStruct((B, L), jnp.int32),
            jax.ShapeDtypeStruct((B, L), jnp.int32),
        ],
    )(k1, cm)

    karr = jnp.broadcast_to(k_eff, (L,)).astype(jnp.int32)
    mesh = plsc.VectorSubcoreMesh(
        core_axis_name="c", subcore_axis_name="s",
        num_cores=NC, num_subcores=NS)
    t_sc = pl.kernel(
        functools.partial(_sc_select, width=width, rpw=rpw),
        out_type=jax.ShapeDtypeStruct((B,), jnp.int32),
        mesh=mesh,
        scratch_types=[
            pltpu.VMEM((FETCH * width,), jnp.float32),
            pltpu.VMEM((width + 4 * L,), jnp.int32),
            pltpu.VMEM((rpw * L,), jnp.int32),
            pltpu.VMEM((rpw * L,), jnp.int32),
            pltpu.VMEM((L,), jnp.int32),
            pltpu.VMEM((L,), jnp.int32),
            pltpu.SemaphoreType.DMA,
        ],
        compiler_params=pltpu.CompilerParams(needs_layout_passes=False),
    )(z.reshape(B * width), t0.reshape(B * L), hi0.reshape(B * L), karr)

    t = jnp.where(k_eff <= 0, jnp.int32(0x7F800000), t_sc).reshape(B, 1)

    nsteps = width // wb2
    zm, xhat = pl.pallas_call(
        _mask_decode_kernel,
        grid=(nsteps,),
        in_specs=[
            pl.BlockSpec((B, 1), lambda i: (0, 0)),
            pl.BlockSpec((B, wb2), lambda i: (0, i)),
            pl.BlockSpec((wb2, d_in), lambda i: (i, 0)),
            pl.BlockSpec((1, d_in), lambda i: (0, 0)),
        ],
        out_specs=[
            pl.BlockSpec((B, wb2), lambda i: (0, i)),
            pl.BlockSpec((B, d_in), lambda i: (0, 0)),
        ],
        out_shape=[
            jax.ShapeDtypeStruct((B, width), jnp.float32),
            jax.ShapeDtypeStruct((B, d_in), jnp.float32),
        ],
    )(t, z, W_enc, db2)

    return (zm, xhat)


# speedup vs baseline: 1.5261x; 1.0008x over previous
import functools

import jax
import jax.numpy as jnp
from jax import lax
from jax.experimental import pallas as pl
from jax.experimental.pallas import tpu as pltpu
from jax.experimental.pallas import tpu_sc as plsc

K_STATIC = 64
NC, NS, L = 2, 16, 16
NW = NC * NS
FETCH = 4


def _encode_kernel(x_ref, w_ref, b_ref, z_ref, cm_ref):
    i = pl.program_id(0)
    z = lax.dot_general(
        x_ref[...], w_ref[...], (((1,), (1,)), ((), ())),
        preferred_element_type=jnp.float32)
    z = jnp.maximum(z + b_ref[...], 0.0)
    z_ref[...] = z
    nsl = z.shape[1] // 128
    cm = z[:, 0:128]
    for a in range(1, nsl):
        cm = jnp.maximum(cm, z[:, a * 128:(a + 1) * 128])

    @pl.when(i == 0)
    def _():
        cm_ref[...] = cm

    @pl.when(i != 0)
    def _():
        cm_ref[...] = jnp.maximum(cm_ref[...], cm)


def _bounds_kernel(k_ref, cm_ref, t0_ref, hi_ref, lo_s, hi_s):
    k = k_ref[0]
    bits = lax.bitcast_convert_type(cm_ref[...], jnp.int32)
    rowmax = jnp.max(bits, axis=1, keepdims=True)
    lo_s[...] = jnp.zeros_like(rowmax)
    hi_s[...] = rowmax + 1

    def body(_, carry):
        lo, hi = lo_s[...], hi_s[...]
        mid = lax.shift_right_logical(lo + hi, 1)
        c = jnp.sum((bits >= mid).astype(jnp.int32), axis=1, keepdims=True)
        ge = c >= k
        lo_s[...] = jnp.where(ge, mid, lo)
        hi_s[...] = jnp.where(ge, hi, mid)
        return carry

    lax.fori_loop(0, 18, body, 0)
    t0_ref[...] = jnp.broadcast_to(jnp.minimum(lo_s[...], rowmax),
                                   t0_ref.shape)
    hi_ref[...] = jnp.broadcast_to(rowmax + 1, hi_ref.shape)


def _sc_select(z_ref, t0_ref, hi0_ref, k_ref, out_ref,
               buf, cand, t0v, hiv, kv, outv, sem, *, width, rpw):
    wid = lax.axis_index("s") * NC + lax.axis_index("c")
    base = wid * rpw

    pltpu.async_copy(
        t0_ref.at[pl.ds(pl.multiple_of(base * L, 8), rpw * L)],
        t0v, sem).wait()
    pltpu.async_copy(
        hi0_ref.at[pl.ds(pl.multiple_of(base * L, 8), rpw * L)],
        hiv, sem).wait()
    pltpu.async_copy(k_ref.at[pl.ds(0, L)], kv, sem).wait()
    k_v = kv[...]
    lane = lax.iota(jnp.int32, L)
    zero_v = jnp.full((L,), 0, jnp.int32)

    def row_body(r, carry):
        @pl.when((r & (FETCH - 1)) == 0)
        def _():
            pltpu.async_copy(
                z_ref.at[pl.ds(pl.multiple_of((base + r) * width, 8),
                               FETCH * width)],
                buf, sem).wait()

        off = (r & (FETCH - 1)) * width
        t0spl = t0v[pl.ds(r * L, L)]
        hispl = hiv[pl.ds(r * L, L)]

        def fbody(i, cnt):
            v = buf[pl.ds(off + i * L, L)]
            b = lax.bitcast_convert_type(v, jnp.int32)
            m = b >= t0spl
            plsc.store_compressed(cand.at[pl.ds(cnt, L)], b, mask=m)
            return cnt + plsc.all_reduce_population_count(m)[0]

        cnt = plsc.parallel_loop(
            0, width // L, unroll=16, carry=jnp.int32(0))(fbody)
        cand[pl.ds(cnt, L)] = jnp.full((L,), -1, jnp.int32)
        nv = lax.shift_right_logical(cnt + L - 1, 4)

        def bis_cond(carry):
            _, _, _, found = carry
            return found == 0

        def bis_body(carry):
            lo, hi, t, found = carry
            mid = lax.shift_right_logical(lo + hi, 1)

            def cb(i, acc):
                cv = cand[pl.ds(i * L, L)]
                return acc + plsc.all_reduce_population_count(cv >= mid)

            c = plsc.parallel_loop(0, nv, unroll=4, carry=zero_v)(cb)
            ge = c >= k_v
            lo = jnp.where(ge, mid, lo)
            hi = jnp.where(jnp.logical_not(ge), mid, hi)
            hit = c == k_v
            closed = (hi - lo) <= 1
            t = jnp.where(hit, mid, jnp.where(closed, lo, t))
            found = jnp.where(
                jnp.logical_or(hit, closed), jnp.int32(1), found)[0]
            return lo, hi, t, found

        _, _, t_v, _ = lax.while_loop(
            bis_cond, bis_body, (t0spl, hispl, zero_v, jnp.int32(0)))

        j = r & (L - 1)
        outv[...] = jnp.where(lane == j, t_v, outv[...])

        @pl.when(j == L - 1)
        def _():
            pltpu.async_copy(
                outv,
                out_ref.at[pl.ds(
                    pl.multiple_of(base + ((r >> 4) << 4), 8), L)],
                sem).wait()

        return carry

    lax.fori_loop(0, rpw, row_body, 0)


def _mask_decode_kernel(t_ref, z_ref, w_ref, db_ref, zm_ref, xhat_ref):
    i = pl.program_id(0)
    z = z_ref[...]
    bits = lax.bitcast_convert_type(z, jnp.int32)
    zm = jnp.where(bits >= t_ref[...], z, 0.0)
    zm_ref[...] = zm
    part = lax.dot_general(
        zm, w_ref[...], (((1,), (0,)), ((), ())),
        preferred_element_type=jnp.float32)

    @pl.when(i == 0)
    def _():
        xhat_ref[...] = part + db_ref[...]

    @pl.when(i != 0)
    def _():
        xhat_ref[...] = xhat_ref[...] + part


def kernel(x, W_enc, b_enc, dec_bias, topk):
    B, d_in = x.shape
    width = W_enc.shape[0]
    wb1 = min(512, width)
    wb2 = min(256, width)
    rb = min(512, B)
    rpw = B // NW

    b2 = b_enc.reshape(1, width)
    db2 = dec_bias.reshape(1, d_in)
    k_eff = jnp.clip(jnp.asarray(topk, jnp.int32), 0, K_STATIC)
    k1 = k_eff.reshape(1)

    z, cm = pl.pallas_call(
        _encode_kernel,
        grid=(width // wb1,),
        in_specs=[
            pl.BlockSpec((B, d_in), lambda i: (0, 0)),
            pl.BlockSpec((wb1, d_in), lambda i: (i, 0)),
            pl.BlockSpec((1, wb1), lambda i: (0, i)),
        ],
        out_specs=[
            pl.BlockSpec((B, wb1), lambda i: (0, i)),
            pl.BlockSpec((B, 128), lambda i: (0, 0)),
        ],
        out_shape=[
            jax.ShapeDtypeStruct((B, width), jnp.float32),
            jax.ShapeDtypeStruct((B, 128), jnp.float32),
        ],
    )(x, W_enc, b2)

    t0, hi0 = pl.pallas_call(
        _bounds_kernel,
        grid_spec=pltpu.PrefetchScalarGridSpec(
            num_scalar_prefetch=1,
            grid=(B // rb,),
            in_specs=[pl.BlockSpec((rb, 128), lambda i, k: (i, 0))],
            out_specs=[
                pl.BlockSpec((rb, L), lambda i, k: (i, 0)),
                pl.BlockSpec((rb, L), lambda i, k: (i, 0)),
            ],
            scratch_shapes=[
                pltpu.VMEM((rb, 1), jnp.int32),
                pltpu.VMEM((rb, 1), jnp.int32),
            ],
        ),
        out_shape=[
            jax.ShapeDtypeStruct((B, L), jnp.int32),
            jax.ShapeDtypeStruct((B, L), jnp.int32),
        ],
    )(k1, cm)

    karr = jnp.broadcast_to(k_eff, (L,)).astype(jnp.int32)
    mesh = plsc.VectorSubcoreMesh(
        core_axis_name="c", subcore_axis_name="s",
        num_cores=NC, num_subcores=NS)
    t_sc = pl.kernel(
        functools.partial(_sc_select, width=width, rpw=rpw),
        out_type=jax.ShapeDtypeStruct((B,), jnp.int32),
        mesh=mesh,
        scratch_types=[
            pltpu.VMEM((FETCH * width,), jnp.float32),
            pltpu.VMEM((width + 4 * L,), jnp.int32),
            pltpu.VMEM((rpw * L,), jnp.int32),
            pltpu.VMEM((rpw * L,), jnp.int32),
            pltpu.VMEM((L,), jnp.int32),
            pltpu.VMEM((L,), jnp.int32),
            pltpu.SemaphoreType.DMA,
        ],
        compiler_params=pltpu.CompilerParams(needs_layout_passes=False),
    )(z.reshape(B * width), t0.reshape(B * L), hi0.reshape(B * L), karr)

    t = jnp.where(k_eff <= 0, jnp.int32(0x7F800000), t_sc).reshape(B, 1)

    nsteps = width // wb2
    zm, xhat = pl.pallas_call(
        _mask_decode_kernel,
        grid=(nsteps,),
        in_specs=[
            pl.BlockSpec((B, 1), lambda i: (0, 0)),
            pl.BlockSpec((B, wb2), lambda i: (0, i)),
            pl.BlockSpec((wb2, d_in), lambda i: (i, 0)),
            pl.BlockSpec((1, d_in), lambda i: (0, 0)),
        ],
        out_specs=[
            pl.BlockSpec((B, wb2), lambda i: (0, i)),
            pl.BlockSpec((B, d_in), lambda i: (0, 0)),
        ],
        out_shape=[
            jax.ShapeDtypeStruct((B, width), jnp.float32),
            jax.ShapeDtypeStruct((B, d_in), jnp.float32),
        ],
    )(t, z, W_enc, db2)

    return (zm, xhat)
